# baseline (speedup 1.0000x reference)
import functools

import jax
import jax.numpy as jnp
from jax import lax
from jax.experimental import pallas as pl
from jax.experimental.pallas import tpu as pltpu
from jax.experimental.pallas import tpu_sc as plsc

N = 320000
D = 128
S = 10000
NW = 32
RPW = N // NW
CH = 80
NCH = RPW // CH
NG = CH // 16
NB_PAD = 96
EB = 96
RING = 8
F32 = jnp.float32
I32 = jnp.int32

_mesh = plsc.VectorSubcoreMesh(core_axis_name="c", subcore_axis_name="s")
_params = pltpu.CompilerParams(needs_layout_passes=False)


def _sread(ref, i):
    return ref[pl.ds(i, 16)][0]


def _scan_chunk(ib, base, prev0, pos):
    lanes = lax.iota(I32, 16)
    shift_idx = jnp.maximum(lanes - 1, 0)
    prev = prev0
    ms, css = [], []
    for g in range(NG):
        v = ib[pl.ds(base + g * 16, 16)]
        shifted = jnp.where(lanes == 0, prev, jnp.take(v, shift_idx))
        m = v != shifted
        ms.append(m)
        css.append(jnp.cumsum(m.astype(I32)))
        prev = v[15]
    nb = jnp.int32(0)
    for g in range(NG):
        plsc.store_scatter(pos, [nb + css[g] - 1], lanes + g * 16,
                           mask=ms[g])
        nb = nb + css[g][15]
    cur = pos[pl.ds(nb, 16)]
    pos[pl.ds(nb, 16)] = jnp.where(lanes == 0, CH, cur)
    return nb


def _body(x_hbm, idx_hbm, t_hbm, w_hbm, out_hbm, ib, xb, eb, pos, prevb,
          tb, wb, ring, zbuf, xsem, fsem):
    wid = lax.axis_index("s") * 2 + lax.axis_index("c")
    r0 = pl.multiple_of(wid * RPW, 8)

    pltpu.sync_copy(idx_hbm.at[pl.ds(r0, RPW)], ib.at[pl.ds(0, RPW)])
    pltpu.sync_copy(t_hbm, tb)
    pltpu.sync_copy(w_hbm, wb)
    for j in range(8):
        tv = tb[pl.ds(j * 16, 16)]
        tb[pl.ds(j * 16, 16)] = jnp.clip(tv, 0.0, 1.0)

    @pl.when(wid > 0)
    def _():
        pltpu.sync_copy(idx_hbm.at[pl.ds(pl.multiple_of(r0 - 8, 8), 16)],
                        prevb)

    tile_prev = jnp.where(wid > 0, prevb[pl.ds(0, 16)][7], jnp.int32(-1))
    wv = wb[pl.ds(0, 16)]
    w0, w1, w2, w3, w4 = wv[0], wv[1], wv[2], wv[3], wv[4]

    pinf = jnp.full((16,), 3.4e38, F32)
    ninf = jnp.full((16,), -3.4e38, F32)
    zero = jnp.zeros((16,), F32)
    for r in range(16 * 8):
        zbuf[pl.ds(r * 16, 16)] = zero

    def zfill(lo, hi):
        nfull = jnp.maximum((hi - lo) // 16, 0)

        def f16(i, _):
            pltpu.sync_copy(
                zbuf,
                out_hbm.at[pl.ds(pl.multiple_of((lo + i * 16) * D, 8),
                                 16 * D)])
            return 0

        lax.fori_loop(0, nfull, f16, 0)

        def f1(g, _):
            pltpu.sync_copy(
                zbuf.at[pl.ds(0, D)],
                out_hbm.at[pl.ds(pl.multiple_of(g * D, 8), D)])
            return 0

        lax.fori_loop(lo + nfull * 16, hi, f1, 0)

    def relu_piece(ref, base, lo, hi, acc, bias):
        def row4(i, a):
            r = base + lo + i * 4
            a2 = []
            for j in range(8):
                l0 = ref[r, pl.ds(j * 16, 16)]
                l1 = ref[r + 1, pl.ds(j * 16, 16)]
                l2 = ref[r + 2, pl.ds(j * 16, 16)]
                l3 = ref[r + 3, pl.ds(j * 16, 16)]
                a2.append(a[j] + ((jnp.maximum(l0 - bias[j], 0.0)
                                   + jnp.maximum(l1 - bias[j], 0.0))
                                  + (jnp.maximum(l2 - bias[j], 0.0)
                                     + jnp.maximum(l3 - bias[j], 0.0))))
            return tuple(a2)

        def row1(r, a):
            return tuple(a[j] + jnp.maximum(ref[base + r, pl.ds(j * 16, 16)]
                                            - bias[j], 0.0)
                         for j in range(8))

        n = hi - lo
        acc = lax.fori_loop(0, n // 4, row4, acc)
        return lax.fori_loop(lo + (n // 4) * 4, hi, row1, acc)

    def flush(cur_seg, new_seg, gs, ge, kcur, mn, mx, sm, dma_cnt):
        zfill(cur_seg + 1, new_seg)

        @pl.when(cur_seg != tile_prev)
        def _():
            @pl.when(dma_cnt >= RING)
            def _():
                pltpu.make_async_copy(ring.at[0],
                                      out_hbm.at[pl.ds(0, D)],
                                      fsem).wait()

            cntf = (ge - gs).astype(F32)
            bias = []
            cvec = []
            for j in range(8):
                ttj = tb[pl.ds(j * 16, 16)]
                bias.append(ttj * mx[j] + (1.0 - ttj) * mn[j])
                cvec.append(w0 * cntf + w1 * mn[j] + w2 * mx[j]
                            + w4 * sm[j])
            bias = tuple(bias)

            def warm(_):
                ca = gs // CH
                mid = jnp.minimum(ge, (ca + 1) * CH)
                acc = relu_piece(xb, (ca % 4) * CH - ca * CH, gs, mid,
                                 (zero,) * 8, bias)
                cb = ca + 1
                acc = lax.cond(
                    ge > mid,
                    lambda a: relu_piece(xb, (cb % 4) * CH - cb * CH, mid,
                                         ge, a, bias),
                    lambda a: a, acc)
                return acc

            def cold(_):
                def piece(c, acc):
                    b0 = gs + c * CH
                    ln = jnp.minimum(CH, ge - b0)
                    a0 = jnp.minimum((r0 + b0) // 8 * 8, N - EB)
                    sh = r0 + b0 - a0
                    pltpu.sync_copy(
                        x_hbm.at[pl.ds(pl.multiple_of(a0, 8), EB)], eb)
                    return relu_piece(eb, sh - b0, b0, b0 + ln, acc, bias)

                npc = (ge - gs + CH - 1) // CH
                return lax.fori_loop(0, npc, piece, (zero,) * 8)

            acc = lax.cond(gs < (kcur - 1) * CH, cold, warm, 0)

            slot = dma_cnt & (RING - 1)
            for j in range(8):
                ring[slot, pl.ds(j * 16, 16)] = cvec[j] + w3 * acc[j]
            pltpu.async_copy(
                ring.at[slot],
                out_hbm.at[pl.ds(pl.multiple_of(cur_seg * D, 8), D)], fsem)

        return jnp.where(cur_seg != tile_prev, dma_cnt + 1, dma_cnt)

    def accum_rows(xbase, lo, hi, mn, mx, sm):
        def row4(i, carry):
            mn, mx, sm = carry
            r = xbase + lo + i * 4
            mn2, mx2, sm2 = [], [], []
            for j in range(8):
                l0 = xb[r, pl.ds(j * 16, 16)]
                l1 = xb[r + 1, pl.ds(j * 16, 16)]
                l2 = xb[r + 2, pl.ds(j * 16, 16)]
                l3 = xb[r + 3, pl.ds(j * 16, 16)]
                mn2.append(jnp.minimum(mn[j], jnp.minimum(
                    jnp.minimum(l0, l1), jnp.minimum(l2, l3))))
                mx2.append(jnp.maximum(mx[j], jnp.maximum(
                    jnp.maximum(l0, l1), jnp.maximum(l2, l3))))
                sm2.append(sm[j] + ((l0 + l1) + (l2 + l3)))
            return tuple(mn2), tuple(mx2), tuple(sm2)

        def row1(r, carry):
            mn, mx, sm = carry
            mn2, mx2, sm2 = [], [], []
            for j in range(8):
                ld = xb[xbase + r, pl.ds(j * 16, 16)]
                mn2.append(jnp.minimum(mn[j], ld))
                mx2.append(jnp.maximum(mx[j], ld))
                sm2.append(sm[j] + ld)
            return tuple(mn2), tuple(mx2), tuple(sm2)

        n = hi - lo
        mn, mx, sm = lax.fori_loop(0, n // 4, row4, (mn, mx, sm))
        return lax.fori_loop(lo + (n // 4) * 4, hi, row1, (mn, mx, sm))

    pltpu.async_copy(x_hbm.at[pl.ds(r0, CH)], xb.at[pl.ds(0, CH)], xsem)
    pltpu.async_copy(x_hbm.at[pl.ds(pl.multiple_of(r0 + CH, 8), CH)],
                     xb.at[pl.ds(CH, CH)], xsem)

    def chunk(k, carry):
        cur_seg, gs, dma_cnt, mn, mx, sm = carry
        xbase = (k % 4) * CH
        pltpu.make_async_copy(x_hbm.at[pl.ds(0, CH)], xb.at[pl.ds(0, CH)],
                              xsem).wait()

        @pl.when(k + 2 < NCH)
        def _():
            pltpu.async_copy(
                x_hbm.at[pl.ds(pl.multiple_of(r0 + (k + 2) * CH, 8), CH)],
                xb.at[pl.ds(((k + 2) % 4) * CH, CH)], xsem)

        prev0 = jnp.where(k > 0, _sread(ib, jnp.maximum(k * CH - 1, 0)),
                          tile_prev)
        nb = _scan_chunk(ib, k * CH, prev0, pos)
        p0 = jnp.where(nb > 0, _sread(pos, 0), CH)
        mn, mx, sm = accum_rows(xbase, 0, p0, mn, mx, sm)

        def seg(j, c):
            cur_seg, gs, dma_cnt, mn, mx, sm = c
            p_lo = _sread(pos, j)
            p_hi = _sread(pos, j + 1)
            ge = k * CH + p_lo
            new_seg = _sread(ib, ge)
            dma_cnt = flush(cur_seg, new_seg, gs, ge, k, mn, mx, sm,
                            dma_cnt)
            mn, mx, sm = accum_rows(xbase, p_lo, p_hi,
                                    (pinf,) * 8, (ninf,) * 8, (zero,) * 8)
            return new_seg, ge, dma_cnt, mn, mx, sm

        return lax.fori_loop(0, nb, seg,
                             (cur_seg, gs, dma_cnt, mn, mx, sm))

    init = (tile_prev, jnp.int32(0), jnp.int32(0),
            (pinf,) * 8, (ninf,) * 8, (zero,) * 8)
    cur_seg, gs, dma_cnt, mn, mx, sm = lax.fori_loop(0, NCH, chunk, init)

    first_ec = pl.multiple_of(r0 + RPW, 8)

    @pl.when(first_ec < N)
    def _():
        pltpu.sync_copy(idx_hbm.at[pl.ds(pl.multiple_of(first_ec, 8), 16)],
                        prevb)

    nxt_id = jnp.where(first_ec < N, prevb[pl.ds(0, 16)][0], jnp.int32(-1))
    cont0 = (nxt_id == cur_seg) & (first_ec < N)

    def ext_cond(c):
        return c[0]

    def ext_body(c):
        _, kc, mn, mx, sm = c
        ec0 = pl.multiple_of(r0 + kc * CH, 8)
        pltpu.sync_copy(idx_hbm.at[pl.ds(ec0, CH)], ib.at[pl.ds(0, CH)])
        pltpu.sync_copy(x_hbm.at[pl.ds(ec0, CH)],
                        xb.at[pl.ds((kc % 4) * CH, CH)])
        nb = _scan_chunk(ib, 0, cur_seg, pos)
        fp = jnp.where(nb > 0, _sread(pos, 0), CH)
        mn, mx, sm = accum_rows((kc % 4) * CH - kc * CH, kc * CH,
                                kc * CH + fp, mn, mx, sm)
        cont = (nb == 0) & (r0 + (kc + 1) * CH < N)
        return cont, kc + 1, mn, mx, sm

    cont_f, kc_f, mn, mx, sm = lax.while_loop(
        ext_cond, ext_body, (cont0, jnp.int32(NCH), mn, mx, sm))

    nb_l = _scan_chunk(ib, 0, cur_seg, pos)
    fp_l = jnp.where(nb_l > 0, _sread(pos, 0), CH)
    ge_f = jnp.where(kc_f > NCH, (kc_f - 1) * CH + fp_l, RPW)
    kcur_f = jnp.maximum(kc_f - 1, NCH - 1)

    dma_cnt = flush(cur_seg, jnp.where(wid == NW - 1, S, cur_seg + 1),
                    gs, ge_f, kcur_f, mn, mx, sm, dma_cnt)

    def drain(i, _):
        @pl.when(i < jnp.minimum(dma_cnt, RING))
        def _():
            pltpu.make_async_copy(ring.at[0], out_hbm.at[pl.ds(0, D)],
                                  fsem).wait()

        return 0

    lax.fori_loop(0, RING, drain, 0)


_fused = functools.partial(
    pl.kernel,
    out_type=jax.ShapeDtypeStruct((S * D,), F32),
    mesh=_mesh,
    compiler_params=_params,
    scratch_types=[
        pltpu.VMEM((RPW + 16,), I32),
        pltpu.VMEM((4 * CH, D), F32),
        pltpu.VMEM((EB, D), F32),
        pltpu.VMEM((NB_PAD + 16,), I32),
        pltpu.VMEM((16,), I32),
        pltpu.VMEM((D,), F32),
        pltpu.VMEM((16,), F32),
        pltpu.VMEM((RING, D), F32),
        pltpu.VMEM((16 * D,), F32),
        pltpu.SemaphoreType.DMA,
        pltpu.SemaphoreType.DMA,
    ],
)(_body)


def kernel(x, batch_idx, max_index, t, W):
    assert x.shape == (N, D)
    idx = batch_idx.astype(I32)
    xf = x.astype(F32)
    t128 = t.astype(F32)
    w5 = jnp.pad(jnp.reshape(W.astype(F32), (5,)), (0, 11))
    out = _fused(xf, idx, t128, w5)
    return jnp.reshape(out, (S, D))


# speedup vs baseline: 1.0430x; 1.0430x over previous
import functools

import jax
import jax.numpy as jnp
from jax import lax
from jax.experimental import pallas as pl
from jax.experimental.pallas import tpu as pltpu
from jax.experimental.pallas import tpu_sc as plsc

N = 320000
D = 128
S = 10000
NW = 32
RPW = N // NW
CH = 80
NCH = RPW // CH
NG = CH // 16
NB_PAD = 96
EB = 96
RING = 8
F32 = jnp.float32
I32 = jnp.int32

_mesh = plsc.VectorSubcoreMesh(core_axis_name="c", subcore_axis_name="s")
_params = pltpu.CompilerParams(needs_layout_passes=False)


def _sread(ref, i):
    return ref[pl.ds(i, 16)][0]


def _scan_chunk(ib, base, prev0, pos):
    lanes = lax.iota(I32, 16)
    shift_idx = jnp.maximum(lanes - 1, 0)
    prev = prev0
    ms, css = [], []
    for g in range(NG):
        v = ib[pl.ds(base + g * 16, 16)]
        shifted = jnp.where(lanes == 0, prev, jnp.take(v, shift_idx))
        m = v != shifted
        ms.append(m)
        css.append(jnp.cumsum(m.astype(I32)))
        prev = v[15]
    nb = jnp.int32(0)
    for g in range(NG):
        plsc.store_scatter(pos, [nb + css[g] - 1], lanes + g * 16,
                           mask=ms[g])
        nb = nb + css[g][15]
    cur = pos[pl.ds(nb, 16)]
    pos[pl.ds(nb, 16)] = jnp.where(lanes == 0, CH, cur)
    return nb


def _body(x_hbm, idx_hbm, t_hbm, w_hbm, out_hbm, ib, xb, eb, pos, prevb,
          tb, wb, ring, zbuf, xsem, fsem):
    wid = lax.axis_index("s") * 2 + lax.axis_index("c")
    r0 = pl.multiple_of(wid * RPW, 8)

    pltpu.sync_copy(idx_hbm.at[pl.ds(r0, RPW)], ib.at[pl.ds(0, RPW)])
    pltpu.sync_copy(t_hbm, tb)
    pltpu.sync_copy(w_hbm, wb)
    for j in range(8):
        tv = tb[pl.ds(j * 16, 16)]
        tb[pl.ds(j * 16, 16)] = jnp.clip(tv, 0.0, 1.0)

    @pl.when(wid > 0)
    def _():
        pltpu.sync_copy(idx_hbm.at[pl.ds(pl.multiple_of(r0 - 8, 8), 16)],
                        prevb)

    tile_prev = jnp.where(wid > 0, prevb[pl.ds(0, 16)][7], jnp.int32(-1))
    wv = wb[pl.ds(0, 16)]
    w0, w1, w2, w3, w4 = wv[0], wv[1], wv[2], wv[3], wv[4]

    pinf = jnp.full((16,), 3.4e38, F32)
    ninf = jnp.full((16,), -3.4e38, F32)
    zero = jnp.zeros((16,), F32)
    for r in range(16 * 8):
        zbuf[pl.ds(r * 16, 16)] = zero

    def zfill(lo, hi):
        nfull = jnp.maximum((hi - lo) // 16, 0)

        def f16(i, _):
            pltpu.sync_copy(
                zbuf,
                out_hbm.at[pl.ds(pl.multiple_of((lo + i * 16) * D, 8),
                                 16 * D)])
            return 0

        lax.fori_loop(0, nfull, f16, 0)

        def f1(g, _):
            pltpu.sync_copy(
                zbuf.at[pl.ds(0, D)],
                out_hbm.at[pl.ds(pl.multiple_of(g * D, 8), D)])
            return 0

        lax.fori_loop(lo + nfull * 16, hi, f1, 0)

    def relu_piece(ref, base, lo, hi, acc, bias):
        def row2(i, a):
            r = base + lo + i * 2
            a2 = []
            for j in range(8):
                ld0 = ref[r, pl.ds(j * 16, 16)]
                ld1 = ref[r + 1, pl.ds(j * 16, 16)]
                a2.append(a[j] + (jnp.maximum(ld0 - bias[j], 0.0)
                                  + jnp.maximum(ld1 - bias[j], 0.0)))
            return tuple(a2)

        n = hi - lo
        acc = lax.fori_loop(0, n // 2, row2, acc)

        def tail(a):
            r = base + hi - 1
            return tuple(a[j] + jnp.maximum(ref[r, pl.ds(j * 16, 16)]
                                            - bias[j], 0.0)
                         for j in range(8))

        return lax.cond(n & 1, tail, lambda a: a, acc)

    def flush(cur_seg, new_seg, gs, ge, kcur, mn, mx, sm, dma_cnt):
        zfill(cur_seg + 1, new_seg)

        @pl.when(cur_seg != tile_prev)
        def _():
            @pl.when(dma_cnt >= RING)
            def _():
                pltpu.make_async_copy(ring.at[0],
                                      out_hbm.at[pl.ds(0, D)],
                                      fsem).wait()

            cntf = (ge - gs).astype(F32)
            bias = []
            cvec = []
            for j in range(8):
                ttj = tb[pl.ds(j * 16, 16)]
                bias.append(ttj * mx[j] + (1.0 - ttj) * mn[j])
                cvec.append(w0 * cntf + w1 * mn[j] + w2 * mx[j]
                            + w4 * sm[j])
            bias = tuple(bias)

            def warm(_):
                ca = gs // CH
                mid = jnp.minimum(ge, (ca + 1) * CH)
                acc = relu_piece(xb, (ca % 4) * CH - ca * CH, gs, mid,
                                 (zero,) * 8, bias)
                cb = ca + 1
                acc = lax.cond(
                    ge > mid,
                    lambda a: relu_piece(xb, (cb % 4) * CH - cb * CH, mid,
                                         ge, a, bias),
                    lambda a: a, acc)
                return acc

            def cold(_):
                def piece(c, acc):
                    b0 = gs + c * CH
                    ln = jnp.minimum(CH, ge - b0)
                    a0 = jnp.minimum((r0 + b0) // 8 * 8, N - EB)
                    sh = r0 + b0 - a0
                    pltpu.sync_copy(
                        x_hbm.at[pl.ds(pl.multiple_of(a0, 8), EB)], eb)
                    return relu_piece(eb, sh - b0, b0, b0 + ln, acc, bias)

                npc = (ge - gs + CH - 1) // CH
                return lax.fori_loop(0, npc, piece, (zero,) * 8)

            acc = lax.cond(gs < (kcur - 1) * CH, cold, warm, 0)

            slot = dma_cnt & (RING - 1)
            for j in range(8):
                ring[slot, pl.ds(j * 16, 16)] = cvec[j] + w3 * acc[j]
            pltpu.async_copy(
                ring.at[slot],
                out_hbm.at[pl.ds(pl.multiple_of(cur_seg * D, 8), D)], fsem)

        return jnp.where(cur_seg != tile_prev, dma_cnt + 1, dma_cnt)

    def accum_rows(xbase, lo, hi, mn, mx, sm):
        def row2(i, carry):
            mn, mx, sm = carry
            r = xbase + lo + i * 2
            mn2, mx2, sm2 = [], [], []
            for j in range(8):
                ld0 = xb[r, pl.ds(j * 16, 16)]
                ld1 = xb[r + 1, pl.ds(j * 16, 16)]
                mn2.append(jnp.minimum(mn[j], jnp.minimum(ld0, ld1)))
                mx2.append(jnp.maximum(mx[j], jnp.maximum(ld0, ld1)))
                sm2.append(sm[j] + (ld0 + ld1))
            return tuple(mn2), tuple(mx2), tuple(sm2)

        n = hi - lo
        mn, mx, sm = lax.fori_loop(0, n // 2, row2, (mn, mx, sm))

        def tail(carry):
            mn, mx, sm = carry
            r = xbase + hi - 1
            mn2, mx2, sm2 = [], [], []
            for j in range(8):
                ld = xb[r, pl.ds(j * 16, 16)]
                mn2.append(jnp.minimum(mn[j], ld))
                mx2.append(jnp.maximum(mx[j], ld))
                sm2.append(sm[j] + ld)
            return tuple(mn2), tuple(mx2), tuple(sm2)

        return lax.cond(n & 1, tail, lambda c: c, (mn, mx, sm))

    pltpu.async_copy(x_hbm.at[pl.ds(r0, CH)], xb.at[pl.ds(0, CH)], xsem)
    pltpu.async_copy(x_hbm.at[pl.ds(pl.multiple_of(r0 + CH, 8), CH)],
                     xb.at[pl.ds(CH, CH)], xsem)

    def chunk(k, carry):
        cur_seg, gs, dma_cnt, mn, mx, sm = carry
        xbase = (k % 4) * CH
        pltpu.make_async_copy(x_hbm.at[pl.ds(0, CH)], xb.at[pl.ds(0, CH)],
                              xsem).wait()

        @pl.when(k + 2 < NCH)
        def _():
            pltpu.async_copy(
                x_hbm.at[pl.ds(pl.multiple_of(r0 + (k + 2) * CH, 8), CH)],
                xb.at[pl.ds(((k + 2) % 4) * CH, CH)], xsem)

        prev0 = jnp.where(k > 0, _sread(ib, jnp.maximum(k * CH - 1, 0)),
                          tile_prev)
        nb = _scan_chunk(ib, k * CH, prev0, pos)
        p0 = jnp.where(nb > 0, _sread(pos, 0), CH)
        mn, mx, sm = accum_rows(xbase, 0, p0, mn, mx, sm)

        def seg(j, c):
            cur_seg, gs, dma_cnt, mn, mx, sm = c
            p_lo = _sread(pos, j)
            p_hi = _sread(pos, j + 1)
            ge = k * CH + p_lo
            new_seg = _sread(ib, ge)
            dma_cnt = flush(cur_seg, new_seg, gs, ge, k, mn, mx, sm,
                            dma_cnt)
            mn, mx, sm = accum_rows(xbase, p_lo, p_hi,
                                    (pinf,) * 8, (ninf,) * 8, (zero,) * 8)
            return new_seg, ge, dma_cnt, mn, mx, sm

        return lax.fori_loop(0, nb, seg,
                             (cur_seg, gs, dma_cnt, mn, mx, sm))

    init = (tile_prev, jnp.int32(0), jnp.int32(0),
            (pinf,) * 8, (ninf,) * 8, (zero,) * 8)
    cur_seg, gs, dma_cnt, mn, mx, sm = lax.fori_loop(0, NCH, chunk, init)

    first_ec = pl.multiple_of(r0 + RPW, 8)

    @pl.when(first_ec < N)
    def _():
        pltpu.sync_copy(idx_hbm.at[pl.ds(pl.multiple_of(first_ec, 8), 16)],
                        prevb)

    nxt_id = jnp.where(first_ec < N, prevb[pl.ds(0, 16)][0], jnp.int32(-1))
    cont0 = (nxt_id == cur_seg) & (first_ec < N)

    def ext_cond(c):
        return c[0]

    def ext_body(c):
        _, kc, mn, mx, sm = c
        ec0 = pl.multiple_of(r0 + kc * CH, 8)
        pltpu.sync_copy(idx_hbm.at[pl.ds(ec0, CH)], ib.at[pl.ds(0, CH)])
        pltpu.sync_copy(x_hbm.at[pl.ds(ec0, CH)],
                        xb.at[pl.ds((kc % 4) * CH, CH)])
        nb = _scan_chunk(ib, 0, cur_seg, pos)
        fp = jnp.where(nb > 0, _sread(pos, 0), CH)
        mn, mx, sm = accum_rows((kc % 4) * CH - kc * CH, kc * CH,
                                kc * CH + fp, mn, mx, sm)
        cont = (nb == 0) & (r0 + (kc + 1) * CH < N)
        return cont, kc + 1, mn, mx, sm

    cont_f, kc_f, mn, mx, sm = lax.while_loop(
        ext_cond, ext_body, (cont0, jnp.int32(NCH), mn, mx, sm))

    nb_l = _scan_chunk(ib, 0, cur_seg, pos)
    fp_l = jnp.where(nb_l > 0, _sread(pos, 0), CH)
    ge_f = jnp.where(kc_f > NCH, (kc_f - 1) * CH + fp_l, RPW)
    kcur_f = jnp.maximum(kc_f - 1, NCH - 1)

    dma_cnt = flush(cur_seg, jnp.where(wid == NW - 1, S, cur_seg + 1),
                    gs, ge_f, kcur_f, mn, mx, sm, dma_cnt)

    def drain(i, _):
        @pl.when(i < jnp.minimum(dma_cnt, RING))
        def _():
            pltpu.make_async_copy(ring.at[0], out_hbm.at[pl.ds(0, D)],
                                  fsem).wait()

        return 0

    lax.fori_loop(0, RING, drain, 0)


_fused = functools.partial(
    pl.kernel,
    out_type=jax.ShapeDtypeStruct((S * D,), F32),
    mesh=_mesh,
    compiler_params=_params,
    scratch_types=[
        pltpu.VMEM((RPW + 16,), I32),
        pltpu.VMEM((4 * CH, D), F32),
        pltpu.VMEM((EB, D), F32),
        pltpu.VMEM((NB_PAD + 16,), I32),
        pltpu.VMEM((16,), I32),
        pltpu.VMEM((D,), F32),
        pltpu.VMEM((16,), F32),
        pltpu.VMEM((RING, D), F32),
        pltpu.VMEM((16 * D,), F32),
        pltpu.SemaphoreType.DMA,
        pltpu.SemaphoreType.DMA,
    ],
)(_body)


def kernel(x, batch_idx, max_index, t, W):
    assert x.shape == (N, D)
    idx = batch_idx.astype(I32)
    xf = x.astype(F32)
    t128 = t.astype(F32)
    w5 = jnp.pad(jnp.reshape(W.astype(F32), (5,)), (0, 11))
    out = _fused(xf, idx, t128, w5)
    return jnp.reshape(out, (S, D))
